# baseline (speedup 1.0000x reference)
import functools

import jax
import jax.numpy as jnp
from jax import lax
from jax.experimental import pallas as pl
from jax.experimental.pallas import tpu as pltpu

_B = 2
_N = 2048
_IN_DIM = 128
_F = 48
_KS = [0.9, 0.8, 0.7]
_HIDDEN = 512
_NCLS = 10

_NS = [2048, 1843, 1474, 1031]
_BM = 256

_INTERPRET = False


def _eye(n, dtype=jnp.float32):
    r = lax.broadcasted_iota(jnp.int32, (n, n), 0)
    c = lax.broadcasted_iota(jnp.int32, (n, n), 1)
    return (r == c).astype(dtype)


def _xposeT(x):
    m = x.shape[0]
    return lax.dot_general(x, _eye(m), (((0,), (0,)), ((), ())),
                           preferred_element_type=jnp.float32)


def _elu(x):
    return jnp.where(x > 0, x, jnp.exp(x) - 1.0)


def _prep_body(g_ref, a_ref, at_ref, s_ref):
    g = g_ref[0]
    patt = (g != 0).astype(jnp.float32)
    a_ref[0] = patt.astype(jnp.bfloat16)
    at_ref[0] = _xposeT(patt).astype(jnp.bfloat16)
    ones = jnp.ones((1, _N), jnp.float32)
    srow = lax.dot_general(ones, g, (((1,), (1,)), ((), ())),
                           preferred_element_type=jnp.float32)
    s_ref[0] = jnp.broadcast_to(srow, (8, _BM))


def _prep(g):
    grid = (_B, _N // _BM)
    return pl.pallas_call(
        _prep_body,
        grid=grid,
        in_specs=[pl.BlockSpec((1, _BM, _N), lambda b, i: (b, i, 0))],
        out_specs=[
            pl.BlockSpec((1, _BM, _N), lambda b, i: (b, i, 0)),
            pl.BlockSpec((1, _N, _BM), lambda b, i: (b, 0, i)),
            pl.BlockSpec((1, 8, _BM), lambda b, i: (b, 0, i)),
        ],
        out_shape=[
            jax.ShapeDtypeStruct((_B, _N, _N), jnp.bfloat16),
            jax.ShapeDtypeStruct((_B, _N, _N), jnp.bfloat16),
            jax.ShapeDtypeStruct((_B, 8, _N), jnp.float32),
        ],
        interpret=_INTERPRET,
    )(g)


def _gcn_body(n_valid, has_scores, has_add, has_add2, fin, *refs):
    i = pl.program_id(1)
    it = iter(refs)
    p_ref = next(it)
    h_ref = next(it)
    va_ref = next(it)
    s_ref = next(it)
    w_ref = next(it)
    b_ref = next(it)
    pw_ref = next(it) if has_scores else None
    pb_ref = next(it) if has_scores else None
    add_ref = next(it) if has_add else None
    add2_ref = next(it) if has_add2 else None
    out_ref = next(it)
    sc_ref = next(it) if has_scores else None
    out2_ref = next(it) if has_add2 else None

    p = p_ref[0].astype(jnp.float32)
    s = s_ref[0][0:1, :]
    va = va_ref[0][0:1, :]
    rs = va * jnp.where(s > 0, 1.0 / s, 0.0)
    ps = p * rs
    acc = jnp.dot(ps, h_ref[0], preferred_element_type=jnp.float32)
    y = jnp.dot(acc, w_ref[...], preferred_element_type=jnp.float32)
    y = _elu(y + b_ref[0:1, :])
    if has_add:
        y = y + add_ref[0]
    out_ref[0] = y
    if has_add2:
        out2_ref[0] = y + add2_ref[0]
    if has_scores:
        srow = lax.dot_general(pw_ref[...], y, (((0,), (1,)), ((), ())),
                               preferred_element_type=jnp.float32)
        srow = srow + pb_ref[0:1, 0:1]
        sig = 1.0 / (1.0 + jnp.exp(-srow))
        col = lax.broadcasted_iota(jnp.int32, (1, _BM), 1) + i * _BM
        sig = jnp.where(col < n_valid, sig, -1.0)
        sc_ref[0] = jnp.broadcast_to(sig, (8, _BM))


def _gcn(P, h, va, s, W, b, *, n_valid, pw=None, pb=None, add=None, add2=None):
    fin = h.shape[-1]
    has_scores = pw is not None
    has_add = add is not None
    has_add2 = add2 is not None
    grid = (_B, _N // _BM)
    in_specs = [
        pl.BlockSpec((1, _BM, _N), lambda b_, i: (b_, i, 0)),
        pl.BlockSpec((1, _N, fin), lambda b_, i: (b_, 0, 0)),
        pl.BlockSpec((1, 8, _N), lambda b_, i: (b_, 0, 0)),
        pl.BlockSpec((1, 8, _N), lambda b_, i: (b_, 0, 0)),
        pl.BlockSpec((fin, _F), lambda b_, i: (0, 0)),
        pl.BlockSpec((8, _F), lambda b_, i: (0, 0)),
    ]
    args = [P, h, va, s, W, b]
    if has_scores:
        in_specs += [pl.BlockSpec((_F, 1), lambda b_, i: (0, 0)),
                     pl.BlockSpec((8, 1), lambda b_, i: (0, 0))]
        args += [pw, pb]
    if has_add:
        in_specs.append(pl.BlockSpec((1, _BM, _F), lambda b_, i: (b_, i, 0)))
        args.append(add)
    if has_add2:
        in_specs.append(pl.BlockSpec((1, _BM, _F), lambda b_, i: (b_, i, 0)))
        args.append(add2)
    out_specs = [pl.BlockSpec((1, _BM, _F), lambda b_, i: (b_, i, 0))]
    out_shape = [jax.ShapeDtypeStruct((_B, _N, _F), jnp.float32)]
    if has_scores:
        out_specs.append(pl.BlockSpec((1, 8, _BM), lambda b_, i: (b_, 0, i)))
        out_shape.append(jax.ShapeDtypeStruct((_B, 8, _N), jnp.float32))
    if has_add2:
        out_specs.append(pl.BlockSpec((1, _BM, _F), lambda b_, i: (b_, i, 0)))
        out_shape.append(jax.ShapeDtypeStruct((_B, _N, _F), jnp.float32))
    res = pl.pallas_call(
        functools.partial(_gcn_body, n_valid, has_scores, has_add, has_add2, fin),
        grid=grid, in_specs=in_specs, out_specs=out_specs, out_shape=out_shape,
        interpret=_INTERPRET,
    )(*args)
    return res if (has_scores or has_add2) else res[0]


def _twohop_body(kk, d_ref, e_ref, p_ref, pt_ref, s_ref):
    i = pl.program_id(1)
    d = d_ref[0]
    e = e_ref[0]
    c = lax.dot_general(d, e, (((1,), (1,)), ((), ())),
                        preferred_element_type=jnp.float32)
    row = lax.broadcasted_iota(jnp.int32, (_BM, _N), 0) + i * _BM
    col = lax.broadcasted_iota(jnp.int32, (_BM, _N), 1)
    patt = jnp.where((c != 0) & (row < kk) & (col < kk), 1.0, 0.0)
    p_ref[0] = patt.astype(jnp.bfloat16)
    pt_ref[0] = _xposeT(patt).astype(jnp.bfloat16)
    ones = jnp.ones((1, _N), jnp.float32)
    srow = lax.dot_general(ones, patt, (((1,), (1,)), ((), ())),
                           preferred_element_type=jnp.float32)
    s_ref[0] = jnp.broadcast_to(srow, (8, _BM))


def _twohop(D, E, kk):
    grid = (_B, _N // _BM)
    return pl.pallas_call(
        functools.partial(_twohop_body, kk),
        grid=grid,
        in_specs=[
            pl.BlockSpec((1, _BM, _N), lambda b, i: (b, i, 0)),
            pl.BlockSpec((1, _N, _N), lambda b, i: (b, 0, 0)),
        ],
        out_specs=[
            pl.BlockSpec((1, _BM, _N), lambda b, i: (b, i, 0)),
            pl.BlockSpec((1, _N, _BM), lambda b, i: (b, 0, i)),
            pl.BlockSpec((1, 8, _BM), lambda b, i: (b, 0, i)),
        ],
        out_shape=[
            jax.ShapeDtypeStruct((_B, _N, _N), jnp.bfloat16),
            jax.ShapeDtypeStruct((_B, _N, _N), jnp.bfloat16),
            jax.ShapeDtypeStruct((_B, 8, _N), jnp.float32),
        ],
        interpret=_INTERPRET,
    )(D, E)


def _readout_body(hs_ref, out_ref):
    seg = pl.program_id(1)
    i = pl.program_id(2)
    n_valid = jnp.where(seg == 0, _NS[2], jnp.where(seg == 1, _NS[1], _NS[0]))
    x = hs_ref[0, 0]
    row = lax.broadcasted_iota(jnp.int32, (_BM, _F), 0) + i * _BM
    mask = row < n_valid
    bmax = jnp.max(jnp.where(mask, x, -jnp.inf), axis=0, keepdims=True)
    bsum = jnp.sum(jnp.where(mask, x, 0.0), axis=0, keepdims=True)

    @pl.when(i == 0)
    def _init():
        out_ref[0, 0, 0:1, :] = bmax
        out_ref[0, 0, 1:2, :] = bsum

    @pl.when(i > 0)
    def _acc():
        out_ref[0, 0, 0:1, :] = jnp.maximum(out_ref[0, 0, 0:1, :], bmax)
        out_ref[0, 0, 1:2, :] = out_ref[0, 0, 1:2, :] + bsum

    @pl.when(i == (_N // _BM) - 1)
    def _fin():
        out_ref[0, 0, 2:3, :] = out_ref[0, 0, 1:2, :] / n_valid.astype(jnp.float32)


def _readout(hstack):
    grid = (_B, 4, _N // _BM)
    return pl.pallas_call(
        _readout_body,
        grid=grid,
        in_specs=[pl.BlockSpec((1, 1, _BM, _F), lambda b, s, i: (b, s, i, 0))],
        out_specs=pl.BlockSpec((1, 1, 8, _F), lambda b, s, i: (b, s, 0, 0)),
        out_shape=jax.ShapeDtypeStruct((_B, 4, 8, _F), jnp.float32),
        interpret=_INTERPRET,
    )(hstack)


def _cls_body(e_ref, w1_ref, b1_ref, w2_ref, b2_ref, o_ref):
    x = jnp.dot(e_ref[...], w1_ref[...], preferred_element_type=jnp.float32)
    x = _elu(x + b1_ref[0:1, :])
    y = jnp.dot(x, w2_ref[...], preferred_element_type=jnp.float32)
    y = y + b2_ref[0:1, :]
    m = jnp.max(y, axis=1, keepdims=True)
    z = y - m
    lse = jnp.log(jnp.sum(jnp.exp(z), axis=1, keepdims=True))
    o_ref[...] = z - lse


def _classifier(emb, w1, b1, w2, b2):
    return pl.pallas_call(
        _cls_body,
        out_shape=jax.ShapeDtypeStruct((_B, _NCLS), jnp.float32),
        interpret=_INTERPRET,
    )(emb, w1, b1, w2, b2)


def _rep8(v):
    out = jnp.zeros((_B, _N), v.dtype).at[:, : v.shape[1]].set(v)
    return jnp.broadcast_to(out[:, None, :], (_B, 8, _N))


def _pad_rows(x, rows=_N):
    pad = rows - x.shape[1]
    if pad == 0:
        return x
    return jnp.pad(x, ((0, 0), (0, pad), (0, 0)))


def kernel(gs, hs, params):
    p = params
    ones_vec = jnp.ones((_B, 8, _N), jnp.float32)

    A0, A0T, s0 = _prep(gs)
    s0 = _rep8(s0[:, 0, :])
    b_s = jnp.broadcast_to(p["s_gcn"]["b"][None, :], (8, _F))

    h = _gcn(gs, hs, ones_vec, s0, p["s_gcn"]["w"], b_s, n_valid=_N)
    org_h = h

    Ps, PTs, ss, vas = [None] * 4, [None] * 4, [None] * 4, [None] * 4
    Ps[0], PTs[0], ss[0], vas[0] = None, None, s0, ones_vec
    down, idxs, valss = [], [], []
    cur_h, cur_va = h, ones_vec
    cur_s = s0
    for lvl in range(3):
        n, kk = _NS[lvl], _NS[lvl + 1]
        Plvl = gs if lvl == 0 else Ps[lvl]
        bd = jnp.broadcast_to(p["down"][lvl]["b"][None, :], (8, _F))
        pw = p["pool"][lvl]["w"]
        pb = jnp.broadcast_to(p["pool"][lvl]["b"][None, :], (8, 1))
        hd, sc = _gcn(Plvl, cur_h, cur_va, cur_s, p["down"][lvl]["w"], bd,
                      n_valid=n, pw=pw, pb=pb)
        down.append(hd)

        scores = sc[:, 0, :]
        vals, idx = jax.vmap(lambda x: lax.top_k(x, kk))(scores)
        idxs.append(idx)
        valss.append(vals)

        src = A0 if lvl == 0 else Ps[lvl]
        srcT = A0T if lvl == 0 else PTs[lvl]
        D = _pad_rows(jnp.take_along_axis(src, idx[:, :, None], axis=1))
        E = _pad_rows(jnp.take_along_axis(srcT, idx[:, :, None], axis=1))
        Hsel = _pad_rows(jnp.take_along_axis(hd, idx[:, :, None], axis=1))

        Pn, PTn, sn = _twohop(D, E, kk)
        Ps[lvl + 1], PTs[lvl + 1], ss[lvl + 1] = Pn, PTn, sn
        vas[lvl + 1] = _rep8(vals)
        cur_h, cur_va, cur_s = Hsel, vas[lvl + 1], sn

    bb = jnp.broadcast_to(p["bottom"]["b"][None, :], (8, _F))
    hb = _gcn(Ps[3], cur_h, cur_va, ss[3], p["bottom"]["w"], bb, n_valid=_NS[3])

    hs_out = []
    cur = hb
    for i in range(3):
        up = 2 - i
        n, kk = _NS[up], _NS[up + 1]
        idx = idxs[up]
        u = jax.vmap(lambda ix, x: jnp.zeros((_N, _F), x.dtype).at[ix].set(x[: ix.shape[0]]))(idx, cur)
        Plvl = gs if up == 0 else Ps[up]
        bu = jnp.broadcast_to(p["up"][i]["b"][None, :], (8, _F))
        if up == 0:
            h_u, h_fin = _gcn(Plvl, u, ones_vec, ss[up], p["up"][i]["w"], bu,
                              n_valid=n, add=down[up], add2=org_h)
            hs_out.append(h_u)
            hs_out.append(h_fin)
        else:
            h_u = _gcn(Plvl, u, ones_vec, ss[up], p["up"][i]["w"], bu,
                       n_valid=n, add=down[up])
            hs_out.append(h_u)
        cur = h_u

    hstack = jnp.stack(hs_out, axis=1)
    ro = _readout(hstack)
    emb = jnp.concatenate([ro[:, s_, r] for r in (0, 1, 2) for s_ in range(4)],
                          axis=-1)
    b1 = jnp.broadcast_to(p["out1"]["b"][None, :], (8, _HIDDEN))
    b2 = jnp.broadcast_to(p["out2"]["b"][None, :], (8, _NCLS))
    return _classifier(emb, p["out1"]["w"], b1, p["out2"]["w"], b2)


# speedup vs baseline: 1.1388x; 1.1388x over previous
import functools

import jax
import jax.numpy as jnp
from jax import lax
from jax.experimental import pallas as pl
from jax.experimental.pallas import tpu as pltpu

_B = 2
_N = 2048
_IN_DIM = 128
_F = 48
_HIDDEN = 512
_NCLS = 10

_NS = [2048, 1843, 1474, 1031]
_NP = [2048, 1920, 1536, 1152]
_BM = 256


def _bm_for(np_):
    for bm in (384, 256, 128):
        if np_ % bm == 0:
            return bm
    raise ValueError(np_)

_INTERPRET = False


def _eye(n, dtype):
    r = lax.broadcasted_iota(jnp.int32, (n, n), 0)
    c = lax.broadcasted_iota(jnp.int32, (n, n), 1)
    return (r == c).astype(dtype)


def _xposeT_bf16(x_bf16):
    m = x_bf16.shape[0]
    return lax.dot_general(x_bf16, _eye(m, jnp.bfloat16),
                           (((0,), (0,)), ((), ())),
                           preferred_element_type=jnp.float32)


def _elu(x):
    return jnp.where(x > 0, x, jnp.exp(x) - 1.0)


def _prep_body(g_ref, a_ref, at_ref, s_ref):
    g = g_ref[0]
    patt = (g != 0).astype(jnp.bfloat16)
    a_ref[0] = patt
    at_ref[0] = _xposeT_bf16(patt).astype(jnp.bfloat16)
    ones = jnp.ones((1, _N), jnp.float32)
    srow = lax.dot_general(ones, g, (((1,), (1,)), ((), ())),
                           preferred_element_type=jnp.float32)
    s_ref[0] = jnp.broadcast_to(srow, (8, _BM))


def _prep(g):
    grid = (_B, _N // _BM)
    return pl.pallas_call(
        _prep_body,
        grid=grid,
        in_specs=[pl.BlockSpec((1, _BM, _N), lambda b, i: (b, i, 0))],
        out_specs=[
            pl.BlockSpec((1, _BM, _N), lambda b, i: (b, i, 0)),
            pl.BlockSpec((1, _N, _BM), lambda b, i: (b, 0, i)),
            pl.BlockSpec((1, 8, _BM), lambda b, i: (b, 0, i)),
        ],
        out_shape=[
            jax.ShapeDtypeStruct((_B, _N, _N), jnp.bfloat16),
            jax.ShapeDtypeStruct((_B, _N, _N), jnp.bfloat16),
            jax.ShapeDtypeStruct((_B, 8, _N), jnp.float32),
        ],
        interpret=_INTERPRET,
    )(g)


def _gcn_body(bm, n_valid, has_scores, has_add, has_add2, fin, *refs):
    i = pl.program_id(1)
    it = iter(refs)
    p_ref = next(it)
    h_ref = next(it)
    va_ref = next(it)
    s_ref = next(it)
    w_ref = next(it)
    b_ref = next(it)
    pw_ref = next(it) if has_scores else None
    pb_ref = next(it) if has_scores else None
    add_ref = next(it) if has_add else None
    add2_ref = next(it) if has_add2 else None
    out_ref = next(it)
    sc_ref = next(it) if has_scores else None
    out2_ref = next(it) if has_add2 else None

    p = p_ref[0].astype(jnp.float32)
    s = s_ref[0][0:1, :]
    va = va_ref[0][0:1, :]
    rs = va * jnp.where(s > 0, 1.0 / s, 0.0)
    ps = p * rs
    acc = jnp.dot(ps, h_ref[0], preferred_element_type=jnp.float32)
    y = jnp.dot(acc, w_ref[...], preferred_element_type=jnp.float32)
    y = _elu(y + b_ref[0:1, :])
    if has_add:
        y = y + add_ref[0]
    out_ref[0] = y
    if has_add2:
        out2_ref[0] = y + add2_ref[0]
    if has_scores:
        srow = lax.dot_general(pw_ref[...], y, (((0,), (1,)), ((), ())),
                               preferred_element_type=jnp.float32)
        srow = srow + pb_ref[0:1, 0:1]
        sig = 1.0 / (1.0 + jnp.exp(-srow))
        col = lax.broadcasted_iota(jnp.int32, (1, bm), 1) + i * bm
        sig = jnp.where(col < n_valid, sig, -1.0)
        sc_ref[0] = jnp.broadcast_to(sig, (8, bm))


def _gcn(P, h, va, s, W, b, *, n_valid, pw=None, pb=None, add=None, add2=None):
    np_ = P.shape[-1]
    fin = h.shape[-1]
    has_scores = pw is not None
    has_add = add is not None
    has_add2 = add2 is not None
    bm = _bm_for(np_)
    grid = (_B, np_ // bm)
    in_specs = [
        pl.BlockSpec((1, bm, np_), lambda b_, i: (b_, i, 0)),
        pl.BlockSpec((1, np_, fin), lambda b_, i: (b_, 0, 0)),
        pl.BlockSpec((1, 8, np_), lambda b_, i: (b_, 0, 0)),
        pl.BlockSpec((1, 8, np_), lambda b_, i: (b_, 0, 0)),
        pl.BlockSpec((fin, _F), lambda b_, i: (0, 0)),
        pl.BlockSpec((8, _F), lambda b_, i: (0, 0)),
    ]
    args = [P, h, va, s, W, b]
    if has_scores:
        in_specs += [pl.BlockSpec((_F, 1), lambda b_, i: (0, 0)),
                     pl.BlockSpec((8, 1), lambda b_, i: (0, 0))]
        args += [pw, pb]
    if has_add:
        in_specs.append(pl.BlockSpec((1, bm, _F), lambda b_, i: (b_, i, 0)))
        args.append(add)
    if has_add2:
        in_specs.append(pl.BlockSpec((1, bm, _F), lambda b_, i: (b_, i, 0)))
        args.append(add2)
    out_specs = [pl.BlockSpec((1, bm, _F), lambda b_, i: (b_, i, 0))]
    out_shape = [jax.ShapeDtypeStruct((_B, np_, _F), jnp.float32)]
    if has_scores:
        out_specs.append(pl.BlockSpec((1, 8, bm), lambda b_, i: (b_, 0, i)))
        out_shape.append(jax.ShapeDtypeStruct((_B, 8, np_), jnp.float32))
    if has_add2:
        out_specs.append(pl.BlockSpec((1, bm, _F), lambda b_, i: (b_, i, 0)))
        out_shape.append(jax.ShapeDtypeStruct((_B, np_, _F), jnp.float32))
    res = pl.pallas_call(
        functools.partial(_gcn_body, bm, n_valid, has_scores, has_add,
                          has_add2, fin),
        grid=grid, in_specs=in_specs, out_specs=out_specs, out_shape=out_shape,
        interpret=_INTERPRET,
    )(*args)
    return res if (has_scores or has_add2) else res[0]


def _twohop_body(bm, kk, kkp, want_t, d_ref, e_ref, *out_refs):
    p_ref = out_refs[0]
    s_ref = out_refs[-1]
    i = pl.program_id(1)
    d = d_ref[0]
    e = e_ref[0]
    c = lax.dot_general(d, e, (((1,), (1,)), ((), ())),
                        preferred_element_type=jnp.float32)
    row = lax.broadcasted_iota(jnp.int32, (bm, kkp), 0) + i * bm
    col = lax.broadcasted_iota(jnp.int32, (bm, kkp), 1)
    pattf = jnp.where((c != 0) & (row < kk) & (col < kk), 1.0, 0.0)
    patt = pattf.astype(jnp.bfloat16)
    p_ref[0] = patt
    if want_t:
        out_refs[1][0] = _xposeT_bf16(patt).astype(jnp.bfloat16)
    ones = jnp.ones((1, kkp), jnp.float32)
    srow = lax.dot_general(ones, pattf, (((1,), (1,)), ((), ())),
                           preferred_element_type=jnp.float32)
    s_ref[0] = jnp.broadcast_to(srow, (8, bm))


def _twohop(D, E, kk, kkp, want_t=True):
    npl = D.shape[-1]
    bm = _bm_for(kkp)
    grid = (_B, kkp // bm)
    out_specs = [pl.BlockSpec((1, bm, kkp), lambda b, i: (b, i, 0))]
    out_shape = [jax.ShapeDtypeStruct((_B, kkp, kkp), jnp.bfloat16)]
    if want_t:
        out_specs.append(pl.BlockSpec((1, kkp, bm), lambda b, i: (b, 0, i)))
        out_shape.append(jax.ShapeDtypeStruct((_B, kkp, kkp), jnp.bfloat16))
    out_specs.append(pl.BlockSpec((1, 8, bm), lambda b, i: (b, 0, i)))
    out_shape.append(jax.ShapeDtypeStruct((_B, 8, kkp), jnp.float32))
    return pl.pallas_call(
        functools.partial(_twohop_body, bm, kk, kkp, want_t),
        grid=grid,
        in_specs=[
            pl.BlockSpec((1, bm, npl), lambda b, i: (b, i, 0)),
            pl.BlockSpec((1, kkp, npl), lambda b, i: (b, 0, 0)),
        ],
        out_specs=out_specs,
        out_shape=out_shape,
        interpret=_INTERPRET,
    )(D, E)


def _readout_body(hs_ref, out_ref):
    seg = pl.program_id(1)
    i = pl.program_id(2)
    n_valid = jnp.where(seg == 0, _NS[2], jnp.where(seg == 1, _NS[1], _NS[0]))
    x = hs_ref[0, 0]
    row = lax.broadcasted_iota(jnp.int32, (_BM, _F), 0) + i * _BM
    mask = row < n_valid
    bmax = jnp.max(jnp.where(mask, x, -jnp.inf), axis=0, keepdims=True)
    bsum = jnp.sum(jnp.where(mask, x, 0.0), axis=0, keepdims=True)

    @pl.when(i == 0)
    def _init():
        out_ref[0, 0, 0:1, :] = bmax
        out_ref[0, 0, 1:2, :] = bsum

    @pl.when(i > 0)
    def _acc():
        out_ref[0, 0, 0:1, :] = jnp.maximum(out_ref[0, 0, 0:1, :], bmax)
        out_ref[0, 0, 1:2, :] = out_ref[0, 0, 1:2, :] + bsum

    @pl.when(i == (_N // _BM) - 1)
    def _fin():
        out_ref[0, 0, 2:3, :] = out_ref[0, 0, 1:2, :] / n_valid.astype(jnp.float32)


def _readout(hstack):
    grid = (_B, 4, _N // _BM)
    return pl.pallas_call(
        _readout_body,
        grid=grid,
        in_specs=[pl.BlockSpec((1, 1, _BM, _F), lambda b, s, i: (b, s, i, 0))],
        out_specs=pl.BlockSpec((1, 1, 8, _F), lambda b, s, i: (b, s, 0, 0)),
        out_shape=jax.ShapeDtypeStruct((_B, 4, 8, _F), jnp.float32),
        interpret=_INTERPRET,
    )(hstack)


def _cls_body(e_ref, w1_ref, b1_ref, w2_ref, b2_ref, o_ref):
    x = jnp.dot(e_ref[...], w1_ref[...], preferred_element_type=jnp.float32)
    x = _elu(x + b1_ref[0:1, :])
    y = jnp.dot(x, w2_ref[...], preferred_element_type=jnp.float32)
    y = y + b2_ref[0:1, :]
    m = jnp.max(y, axis=1, keepdims=True)
    z = y - m
    lse = jnp.log(jnp.sum(jnp.exp(z), axis=1, keepdims=True))
    o_ref[...] = z - lse


def _classifier(emb, w1, b1, w2, b2):
    return pl.pallas_call(
        _cls_body,
        out_shape=jax.ShapeDtypeStruct((_B, _NCLS), jnp.float32),
        interpret=_INTERPRET,
    )(emb, w1, b1, w2, b2)


def _rep8(v, np_):
    out = jnp.zeros((_B, np_), v.dtype).at[:, : v.shape[1]].set(v)
    return jnp.broadcast_to(out[:, None, :], (_B, 8, np_))


def _pad_rows(x, rows):
    pad = rows - x.shape[1]
    if pad == 0:
        return x
    return jnp.pad(x, ((0, 0), (0, pad), (0, 0)))


def kernel(gs, hs, params):
    p = params

    A0, A0T, s0 = _prep(gs)
    b_s = jnp.broadcast_to(p["s_gcn"]["b"][None, :], (8, _F))
    ones0 = jnp.ones((_B, 8, _N), jnp.float32)

    h = _gcn(gs, hs, ones0, s0, p["s_gcn"]["w"], b_s, n_valid=_N)
    org_h = h

    Ps, PTs, ss, vas = [None] * 4, [None] * 4, [None] * 4, [None] * 4
    ss[0], vas[0] = s0, ones0
    down, idxs = [], []
    cur_h = h
    for lvl in range(3):
        n, kk = _NS[lvl], _NS[lvl + 1]
        npl, kkp = _NP[lvl], _NP[lvl + 1]
        Plvl = gs if lvl == 0 else Ps[lvl]
        bd = jnp.broadcast_to(p["down"][lvl]["b"][None, :], (8, _F))
        pw = p["pool"][lvl]["w"]
        pb = jnp.broadcast_to(p["pool"][lvl]["b"][None, :], (8, 1))
        hd, sc = _gcn(Plvl, cur_h, vas[lvl], ss[lvl], p["down"][lvl]["w"], bd,
                      n_valid=n, pw=pw, pb=pb)
        down.append(hd)

        scores = sc[:, 0, :]
        vals, idx = jax.vmap(lambda x: lax.top_k(x, kk))(scores)
        idxs.append(idx)

        src = A0 if lvl == 0 else Ps[lvl]
        srcT = A0T if lvl == 0 else PTs[lvl]
        D = _pad_rows(jnp.take_along_axis(src, idx[:, :, None], axis=1), kkp)
        E = _pad_rows(jnp.take_along_axis(srcT, idx[:, :, None], axis=1), kkp)
        Hsel = _pad_rows(jnp.take_along_axis(hd, idx[:, :, None], axis=1), kkp)

        res = _twohop(D, E, kk, kkp, want_t=(lvl < 2))
        Ps[lvl + 1] = res[0]
        PTs[lvl + 1] = res[1] if lvl < 2 else None
        ss[lvl + 1] = res[-1]
        vas[lvl + 1] = _rep8(vals, kkp)
        cur_h = Hsel

    bb = jnp.broadcast_to(p["bottom"]["b"][None, :], (8, _F))
    hb = _gcn(Ps[3], cur_h, vas[3], ss[3], p["bottom"]["w"], bb, n_valid=_NS[3])

    hs_out = []
    cur = hb
    for i in range(3):
        up = 2 - i
        n, npl = _NS[up], _NP[up]
        kk = _NS[up + 1]
        idx = idxs[up]
        u = jax.vmap(
            lambda ix, x: jnp.zeros((npl, _F), x.dtype).at[ix].set(x[: ix.shape[0]])
        )(idx, cur)
        Plvl = gs if up == 0 else Ps[up]
        ones_l = jnp.ones((_B, 8, npl), jnp.float32)
        bu = jnp.broadcast_to(p["up"][i]["b"][None, :], (8, _F))
        if up == 0:
            h_u, h_fin = _gcn(Plvl, u, ones_l, ss[up], p["up"][i]["w"], bu,
                              n_valid=n, add=down[up], add2=org_h)
            hs_out.append(h_u)
            hs_out.append(h_fin)
        else:
            h_u = _gcn(Plvl, u, ones_l, ss[up], p["up"][i]["w"], bu,
                       n_valid=n, add=down[up])
            hs_out.append(h_u)
        cur = h_u

    hstack = jnp.stack([_pad_rows(x, _N) for x in hs_out], axis=1)
    ro = _readout(hstack)
    emb = jnp.concatenate([ro[:, s_, r] for r in (0, 1, 2) for s_ in range(4)],
                          axis=-1)
    b1 = jnp.broadcast_to(p["out1"]["b"][None, :], (8, _HIDDEN))
    b2 = jnp.broadcast_to(p["out2"]["b"][None, :], (8, _NCLS))
    return _classifier(emb, p["out1"]["w"], b1, p["out2"]["w"], b2)


# speedup vs baseline: 1.4502x; 1.2735x over previous
import functools

import jax
import jax.numpy as jnp
from jax import lax
from jax.experimental import pallas as pl
from jax.experimental.pallas import tpu as pltpu
from jax.experimental.pallas import tpu_sc as plsc

_B = 2
_N = 2048
_IN_DIM = 128
_F = 48
_FP = 128
_HIDDEN = 512
_NCLS = 10

_NS = [2048, 1843, 1474, 1031]
_NP = [2048, 1920, 1536, 1152]
_BM = 256


def _bm_for(np_):
    for bm in (384, 256, 128):
        if np_ % bm == 0:
            return bm
    raise ValueError(np_)

_INTERPRET = False


def _eye(n, dtype):
    r = lax.broadcasted_iota(jnp.int32, (n, n), 0)
    c = lax.broadcasted_iota(jnp.int32, (n, n), 1)
    return (r == c).astype(dtype)


def _xposeT_bf16(x_bf16):
    m = x_bf16.shape[0]
    return lax.dot_general(x_bf16, _eye(m, jnp.bfloat16),
                           (((0,), (0,)), ((), ())),
                           preferred_element_type=jnp.float32)


def _elu(x):
    return jnp.where(x > 0, x, jnp.exp(x) - 1.0)


def _prep_body(g_ref, a_ref, at_ref, s_ref):
    g = g_ref[0]
    pattf = jnp.where(g != 0, 1.0, 0.0)
    a_ref[0] = pattf
    at_ref[0] = _xposeT_bf16(pattf.astype(jnp.bfloat16))
    ones = jnp.ones((1, _N), jnp.float32)
    srow = lax.dot_general(ones, g, (((1,), (1,)), ((), ())),
                           precision=lax.Precision.HIGHEST,
                           preferred_element_type=jnp.float32)
    s_ref[0] = jnp.broadcast_to(srow, (8, _BM))


def _prep(g):
    grid = (_B, _N // _BM)
    return pl.pallas_call(
        _prep_body,
        grid=grid,
        in_specs=[pl.BlockSpec((1, _BM, _N), lambda b, i: (b, i, 0))],
        out_specs=[
            pl.BlockSpec((1, _BM, _N), lambda b, i: (b, i, 0)),
            pl.BlockSpec((1, _N, _BM), lambda b, i: (b, 0, i)),
            pl.BlockSpec((1, 8, _BM), lambda b, i: (b, 0, i)),
        ],
        out_shape=[
            jax.ShapeDtypeStruct((_B, _N, _N), jnp.float32),
            jax.ShapeDtypeStruct((_B, _N, _N), jnp.float32),
            jax.ShapeDtypeStruct((_B, 8, _N), jnp.float32),
        ],
        interpret=_INTERPRET,
    )(g)


def _gcn_body(bm, n_valid, has_scores, has_add, has_add2, fin, *refs):
    i = pl.program_id(1)
    it = iter(refs)
    p_ref = next(it)
    h_ref = next(it)
    va_ref = next(it)
    s_ref = next(it)
    w_ref = next(it)
    b_ref = next(it)
    pw_ref = next(it) if has_scores else None
    pb_ref = next(it) if has_scores else None
    add_ref = next(it) if has_add else None
    add2_ref = next(it) if has_add2 else None
    out_ref = next(it)
    sc_ref = next(it) if has_scores else None
    out2_ref = next(it) if has_add2 else None

    p = p_ref[0].astype(jnp.float32)
    s = s_ref[0][0:1, :]
    va = va_ref[0][0:1, :]
    rs = va * jnp.where(s > 0, 1.0 / s, 0.0)
    ps = p * rs
    acc = jnp.dot(ps, h_ref[0], preferred_element_type=jnp.float32)
    y = jnp.dot(acc, w_ref[...], preferred_element_type=jnp.float32)
    y = _elu(y + b_ref[0:1, :])
    if has_add:
        y = y + add_ref[0]
    out_ref[0] = y
    if has_add2:
        out2_ref[0] = y + add2_ref[0]
    if has_scores:
        srow = lax.dot_general(pw_ref[...], y, (((0,), (1,)), ((), ())),
                               preferred_element_type=jnp.float32)
        srow = srow + pb_ref[0:1, 0:1]
        sig = 1.0 / (1.0 + jnp.exp(-srow))
        col = lax.broadcasted_iota(jnp.int32, (1, bm), 1) + i * bm
        sig = jnp.where(col < n_valid, sig, -1.0)
        sc_ref[0] = jnp.broadcast_to(sig, (8, bm))


def _gcn(P, h, va, s, W, b, *, n_valid, pw=None, pb=None, add=None, add2=None):
    np_ = P.shape[-1]
    fin = h.shape[-1]
    has_scores = pw is not None
    has_add = add is not None
    has_add2 = add2 is not None
    bm = _bm_for(np_)
    grid = (_B, np_ // bm)
    in_specs = [
        pl.BlockSpec((1, bm, np_), lambda b_, i: (b_, i, 0)),
        pl.BlockSpec((1, np_, fin), lambda b_, i: (b_, 0, 0)),
        pl.BlockSpec((1, 8, np_), lambda b_, i: (b_, 0, 0)),
        pl.BlockSpec((1, 8, np_), lambda b_, i: (b_, 0, 0)),
        pl.BlockSpec((fin, _FP), lambda b_, i: (0, 0)),
        pl.BlockSpec((8, _FP), lambda b_, i: (0, 0)),
    ]
    args = [P, h, va, s, W, b]
    if has_scores:
        in_specs += [pl.BlockSpec((_FP, 1), lambda b_, i: (0, 0)),
                     pl.BlockSpec((8, 1), lambda b_, i: (0, 0))]
        args += [pw, pb]
    if has_add:
        in_specs.append(pl.BlockSpec((1, bm, _FP), lambda b_, i: (b_, i, 0)))
        args.append(add)
    if has_add2:
        in_specs.append(pl.BlockSpec((1, bm, _FP), lambda b_, i: (b_, i, 0)))
        args.append(add2)
    out_specs = [pl.BlockSpec((1, bm, _FP), lambda b_, i: (b_, i, 0))]
    out_shape = [jax.ShapeDtypeStruct((_B, np_, _FP), jnp.float32)]
    if has_scores:
        out_specs.append(pl.BlockSpec((1, 8, bm), lambda b_, i: (b_, 0, i)))
        out_shape.append(jax.ShapeDtypeStruct((_B, 8, np_), jnp.float32))
    if has_add2:
        out_specs.append(pl.BlockSpec((1, bm, _FP), lambda b_, i: (b_, i, 0)))
        out_shape.append(jax.ShapeDtypeStruct((_B, np_, _FP), jnp.float32))
    res = pl.pallas_call(
        functools.partial(_gcn_body, bm, n_valid, has_scores, has_add,
                          has_add2, fin),
        grid=grid, in_specs=in_specs, out_specs=out_specs, out_shape=out_shape,
        interpret=_INTERPRET,
    )(*args)
    return res if (has_scores or has_add2) else res[0]


def _twohop_body(bm, kk, kkp, want_t, d_ref, e_ref, *out_refs):
    p_ref = out_refs[0]
    s_ref = out_refs[-1]
    i = pl.program_id(1)
    d = d_ref[0].astype(jnp.bfloat16)
    e = e_ref[0].astype(jnp.bfloat16)
    c = lax.dot_general(d, e, (((1,), (1,)), ((), ())),
                        preferred_element_type=jnp.float32)
    row = lax.broadcasted_iota(jnp.int32, (bm, kkp), 0) + i * bm
    col = lax.broadcasted_iota(jnp.int32, (bm, kkp), 1)
    pattf = jnp.where((c != 0) & (row < kk) & (col < kk), 1.0, 0.0)
    p_ref[0] = pattf
    if want_t:
        out_refs[1][0] = _xposeT_bf16(pattf.astype(jnp.bfloat16))
    ones = jnp.ones((1, kkp), jnp.float32)
    srow = lax.dot_general(ones, pattf, (((1,), (1,)), ((), ())),
                           preferred_element_type=jnp.float32)
    s_ref[0] = jnp.broadcast_to(srow, (8, bm))


def _twohop(D, E, kk, kkp, want_t=True):
    npl = D.shape[-1]
    bm = _bm_for(kkp)
    grid = (_B, kkp // bm)
    out_specs = [pl.BlockSpec((1, bm, kkp), lambda b, i: (b, i, 0))]
    out_shape = [jax.ShapeDtypeStruct((_B, kkp, kkp), jnp.float32)]
    if want_t:
        out_specs.append(pl.BlockSpec((1, kkp, bm), lambda b, i: (b, 0, i)))
        out_shape.append(jax.ShapeDtypeStruct((_B, kkp, kkp), jnp.float32))
    out_specs.append(pl.BlockSpec((1, 8, bm), lambda b, i: (b, 0, i)))
    out_shape.append(jax.ShapeDtypeStruct((_B, 8, kkp), jnp.float32))
    return pl.pallas_call(
        functools.partial(_twohop_body, bm, kk, kkp, want_t),
        grid=grid,
        in_specs=[
            pl.BlockSpec((1, bm, npl), lambda b, i: (b, i, 0)),
            pl.BlockSpec((1, kkp, npl), lambda b, i: (b, 0, 0)),
        ],
        out_specs=out_specs,
        out_shape=out_shape,
        interpret=_INTERPRET,
    )(D, E)


def _select_body(npl, kk, sc_ref, idx_ref, val_ref, rank_ref):
    bm = 128
    s = sc_ref[0, 0:1, :]
    jglob = lax.broadcasted_iota(jnp.int32, (bm, npl), 1)
    ones = jnp.ones((1, npl), jnp.float32)
    iotaf = lax.broadcasted_iota(jnp.int32, (1, npl), 1).astype(jnp.float32)
    eye = _eye(bm, jnp.float32)
    for ib in range(npl // bm):
        sblk = s[:, ib * bm:(ib + 1) * bm]
        siT = lax.dot_general(eye, sblk, (((1,), (1,)), ((), ())),
                              precision=lax.Precision.HIGHEST,
                              preferred_element_type=jnp.float32)
        ig = lax.broadcasted_iota(jnp.int32, (bm, 1), 0) + ib * bm
        keep = (s > siT) | ((s == siT) & (jglob < ig))
        m = jnp.where(keep, 1.0, 0.0)
        cnt = lax.dot_general(ones, m, (((1,), (1,)), ((), ())),
                              precision=lax.Precision.HIGHEST,
                              preferred_element_type=jnp.float32)
        rank_ref[0:1, ib * bm:(ib + 1) * bm] = cnt
    rrow = rank_ref[0:1, :]
    for jb in range(2048 // bm):
        jT = (lax.broadcasted_iota(jnp.int32, (bm, 1), 0) + jb * bm
              ).astype(jnp.float32)
        eq = jnp.where(rrow == jT, 1.0, 0.0)
        inv = lax.dot_general(iotaf, eq, (((1,), (1,)), ((), ())),
                              precision=lax.Precision.HIGHEST,
                              preferred_element_type=jnp.float32)
        v = lax.dot_general(s, eq, (((1,), (1,)), ((), ())),
                            precision=lax.Precision.HIGHEST,
                            preferred_element_type=jnp.float32)
        jrow = lax.broadcasted_iota(jnp.int32, (1, bm), 1) + jb * bm
        idxv = jnp.where(jrow < kk, inv.astype(jnp.int32), npl)
        vv = jnp.where(jrow < kk, v, 0.0)
        idx_ref[0, :, jb * bm:(jb + 1) * bm] = jnp.broadcast_to(idxv, (8, bm))
        val_ref[0, :, jb * bm:(jb + 1) * bm] = jnp.broadcast_to(vv, (8, bm))


def _select(sc, kk):
    npl = sc.shape[-1]
    return pl.pallas_call(
        functools.partial(_select_body, npl, kk),
        grid=(_B,),
        in_specs=[pl.BlockSpec((1, 8, npl), lambda b: (b, 0, 0))],
        out_specs=[pl.BlockSpec((1, 8, 2048), lambda b: (b, 0, 0)),
                   pl.BlockSpec((1, 8, 2048), lambda b: (b, 0, 0))],
        out_shape=[jax.ShapeDtypeStruct((_B, 8, 2048), jnp.int32),
                   jax.ShapeDtypeStruct((_B, 8, 2048), jnp.float32)],
        scratch_shapes=[pltpu.VMEM((8, npl), jnp.float32)],
        interpret=_INTERPRET,
    )(sc)


def _make_gather(npl, kkp, fin):
    nchunk = kkp // 64
    total = _B * nchunk
    iters = -(-total // 32)
    mesh = plsc.VectorSubcoreMesh(core_axis_name="c", subcore_axis_name="s")

    @functools.partial(
        pl.kernel, mesh=mesh,
        out_type=[
            jax.ShapeDtypeStruct((_B, kkp, npl), jnp.float32),
            jax.ShapeDtypeStruct((_B, kkp, npl), jnp.float32),
            jax.ShapeDtypeStruct((_B, kkp, fin), jnp.float32),
        ],
        scratch_types=[
            pltpu.VMEM((64,), jnp.int32),
            pltpu.VMEM((16, npl), jnp.float32),
            pltpu.VMEM((16, npl), jnp.float32),
            pltpu.VMEM((16, fin), jnp.float32),
            pltpu.SemaphoreType.DMA,
        ],
    )
    def gat(idx_hbm, p_hbm, pt_hbm, h_hbm,
            d_out, e_out, hsel_out,
            win, dbuf, ebuf, hbuf, sem):
        wid = lax.axis_index("s") * 2 + lax.axis_index("c")
        for it_ in range(iters):
            c = wid + it_ * 32

            @pl.when(c < total)
            def _gather():
                b = c // nchunk
                r0 = pl.multiple_of((c % nchunk) * 64, 64)
                pltpu.sync_copy(idx_hbm.at[b, 0, pl.ds(r0, 64)], win)
                for q in range(4):
                    off = pl.multiple_of(q * 16, 16)
                    iv = jnp.minimum(win[pl.ds(off, 16)], npl - 1)
                    cd = pltpu.async_copy(p_hbm.at[b].at[iv], dbuf, sem)
                    ce = pltpu.async_copy(pt_hbm.at[b].at[iv], ebuf, sem)
                    ch = pltpu.async_copy(h_hbm.at[b].at[iv], hbuf, sem)
                    cd.wait()
                    pltpu.sync_copy(dbuf, d_out.at[b, pl.ds(r0 + off, 16)])
                    ce.wait()
                    pltpu.sync_copy(ebuf, e_out.at[b, pl.ds(r0 + off, 16)])
                    ch.wait()
                    pltpu.sync_copy(hbuf, hsel_out.at[b, pl.ds(r0 + off, 16)])

    return gat


def _make_scatter(npl, kk, nsrc, fin):
    rows_out = npl + 64
    nz = rows_out // 64
    ziters = -(-nz // 16)
    nchunk = -(-kk // 64)
    iters = -(-nchunk // 16)
    mesh = plsc.VectorSubcoreMesh(core_axis_name="c", subcore_axis_name="s")

    @functools.partial(
        pl.kernel, mesh=mesh,
        out_type=jax.ShapeDtypeStruct((_B, rows_out, fin), jnp.float32),
        scratch_types=[
            pltpu.VMEM((64, fin), jnp.float32),
            pltpu.VMEM((64,), jnp.int32),
            pltpu.VMEM((16, fin), jnp.float32),
            pltpu.SemaphoreType.DMA,
        ],
    )
    def scat(z_hbm, idx_hbm, h_hbm, u_out, zv, win, hbuf, sem):
        b = lax.axis_index("c")
        sid = lax.axis_index("s")
        pltpu.sync_copy(z_hbm, zv)
        for zt in range(ziters):
            zc = sid + zt * 16

            @pl.when(zc < nz)
            def _zero():
                zr0 = pl.multiple_of(zc * 64, 64)
                pltpu.sync_copy(zv, u_out.at[b, pl.ds(zr0, 64)])

        plsc.subcore_barrier()

        for it_ in range(iters):
            c = sid + it_ * 16

            @pl.when(c < nchunk)
            def _scatter():
                r0 = pl.multiple_of(c * 64, 64)
                pltpu.sync_copy(idx_hbm.at[b, 0, pl.ds(r0, 64)], win)
                for q in range(4):
                    off = pl.multiple_of(q * 16, 16)
                    iv = win[pl.ds(off, 16)]
                    pltpu.sync_copy(h_hbm.at[b, pl.ds(r0 + off, 16)], hbuf)
                    pltpu.async_copy(hbuf, u_out.at[b].at[iv], sem).wait()

    return scat


def _readout_body(hs_ref, out_ref):
    seg = pl.program_id(1)
    i = pl.program_id(2)
    n_valid = jnp.where(seg == 0, _NS[2], jnp.where(seg == 1, _NS[1], _NS[0]))
    x = hs_ref[0, 0]
    row = lax.broadcasted_iota(jnp.int32, (_BM, _FP), 0) + i * _BM
    mask = row < n_valid
    bmax = jnp.max(jnp.where(mask, x, -jnp.inf), axis=0, keepdims=True)
    bsum = jnp.sum(jnp.where(mask, x, 0.0), axis=0, keepdims=True)

    @pl.when(i == 0)
    def _init():
        out_ref[0, 0, 0:1, :] = bmax
        out_ref[0, 0, 1:2, :] = bsum

    @pl.when(i > 0)
    def _acc():
        out_ref[0, 0, 0:1, :] = jnp.maximum(out_ref[0, 0, 0:1, :], bmax)
        out_ref[0, 0, 1:2, :] = out_ref[0, 0, 1:2, :] + bsum

    @pl.when(i == (_N // _BM) - 1)
    def _fin():
        out_ref[0, 0, 2:3, :] = out_ref[0, 0, 1:2, :] / n_valid.astype(jnp.float32)


def _readout(hstack):
    grid = (_B, 4, _N // _BM)
    return pl.pallas_call(
        _readout_body,
        grid=grid,
        in_specs=[pl.BlockSpec((1, 1, _BM, _FP), lambda b, s, i: (b, s, i, 0))],
        out_specs=pl.BlockSpec((1, 1, 8, _FP), lambda b, s, i: (b, s, 0, 0)),
        out_shape=jax.ShapeDtypeStruct((_B, 4, 8, _FP), jnp.float32),
        interpret=_INTERPRET,
    )(hstack)


def _cls_body(e_ref, w1_ref, b1_ref, w2_ref, b2_ref, o_ref):
    x = jnp.dot(e_ref[...], w1_ref[...], preferred_element_type=jnp.float32)
    x = _elu(x + b1_ref[0:1, :])
    y = jnp.dot(x, w2_ref[...], preferred_element_type=jnp.float32)
    y = y + b2_ref[0:1, :]
    m = jnp.max(y, axis=1, keepdims=True)
    z = y - m
    lse = jnp.log(jnp.sum(jnp.exp(z), axis=1, keepdims=True))
    o_ref[...] = z - lse


def _classifier(emb, w1, b1, w2, b2):
    return pl.pallas_call(
        _cls_body,
        out_shape=jax.ShapeDtypeStruct((_B, _NCLS), jnp.float32),
        interpret=_INTERPRET,
    )(emb, w1, b1, w2, b2)


def _rep8(v, np_):
    out = jnp.zeros((_B, np_), v.dtype).at[:, : v.shape[1]].set(v)
    return jnp.broadcast_to(out[:, None, :], (_B, 8, np_))


def _pad_rows(x, rows):
    pad = rows - x.shape[1]
    if pad == 0:
        return x
    return jnp.pad(x, ((0, 0), (0, pad), (0, 0)))


def _padW(w):
    return jnp.pad(w, ((0, _FP - w.shape[0]), (0, _FP - w.shape[1])))


def _padb(b):
    return jnp.broadcast_to(jnp.pad(b, (0, _FP - b.shape[0]))[None, :],
                            (8, _FP))


_USE_SC_GATHER = True
_USE_TC_SELECT = True
_USE_SC_SCATTER = True


def _fake_gather(npl, kkp, fin):
    def f(idx8, P, PT, hd):
        idx = jnp.minimum(idx8[:, 0, :kkp], npl - 1)
        D = jnp.take_along_axis(P, idx[:, :, None], axis=1)
        E = jnp.take_along_axis(PT, idx[:, :, None], axis=1)
        Hs = jnp.take_along_axis(hd, idx[:, :, None], axis=1)
        return D, E, Hs
    return f


def _fake_scatter(npl, kk, nsrc, fin):
    def f(z, idx8, cur):
        nch = -(-kk // 64) * 64
        idx = idx8[:, 0, :nch]
        def one(ix, x):
            return jnp.zeros((npl + 64, fin), x.dtype).at[ix].set(x[:nch])
        return jax.vmap(one)(idx, cur)
    return f


def kernel(gs, hs, params):
    p = params

    A0, A0T, s0 = _prep(gs)
    ones0 = jnp.ones((_B, 8, _N), jnp.float32)

    h = _gcn(gs, hs, ones0, s0, _padW(p["s_gcn"]["w"]), _padb(p["s_gcn"]["b"]),
             n_valid=_N)
    org_h = h

    Ps, PTs, ss, vas = [None] * 4, [None] * 4, [None] * 4, [None] * 4
    ss[0], vas[0] = s0, ones0
    down, idxs = [], []
    cur_h = h
    for lvl in range(3):
        n, kk = _NS[lvl], _NS[lvl + 1]
        npl, kkp = _NP[lvl], _NP[lvl + 1]
        Plvl = gs if lvl == 0 else Ps[lvl]
        pw = jnp.pad(p["pool"][lvl]["w"], ((0, _FP - _F), (0, 0)))
        pb = jnp.broadcast_to(p["pool"][lvl]["b"][None, :], (8, 1))
        hd, sc = _gcn(Plvl, cur_h, vas[lvl], ss[lvl],
                      _padW(p["down"][lvl]["w"]), _padb(p["down"][lvl]["b"]),
                      n_valid=n, pw=pw, pb=pb)
        down.append(hd)

        if _USE_TC_SELECT:
            idx, vals = _select(sc, kk)
        else:
            v_k, i_k = jax.vmap(lambda x: lax.top_k(x, kk))(sc[:, 0, :])
            i_p = jnp.pad(i_k, ((0, 0), (0, 2048 - kk)),
                          constant_values=npl)
            v_p = jnp.pad(v_k, ((0, 0), (0, 2048 - kk)))
            idx = jnp.broadcast_to(i_p[:, None, :], (_B, 8, 2048))
            vals = jnp.broadcast_to(v_p[:, None, :], (_B, 8, 2048))
        src = A0 if lvl == 0 else Ps[lvl]
        srcT = A0T if lvl == 0 else PTs[lvl]
        gat = _make_gather if _USE_SC_GATHER else _fake_gather
        D, E, Hsel = gat(npl, kkp, _FP)(idx, src, srcT, hd)
        idxs.append(idx)

        res = _twohop(D, E, kk, kkp, want_t=(lvl < 2))
        Ps[lvl + 1] = res[0]
        PTs[lvl + 1] = res[1] if lvl < 2 else None
        ss[lvl + 1] = res[-1]
        vas[lvl + 1] = vals[:, :, :kkp]
        cur_h = Hsel

    hb = _gcn(Ps[3], cur_h, vas[3], ss[3], _padW(p["bottom"]["w"]),
              _padb(p["bottom"]["b"]), n_valid=_NS[3])

    hs_out = []
    cur = hb
    for i in range(3):
        up = 2 - i
        n, npl = _NS[up], _NP[up]
        kk = _NS[up + 1]
        z = jnp.zeros((64, _FP), jnp.float32)
        sca = _make_scatter if _USE_SC_SCATTER else _fake_scatter
        u = sca(npl, kk, cur.shape[1], _FP)(z, idxs[up], cur)
        Plvl = gs if up == 0 else Ps[up]
        ones_l = jnp.ones((_B, 8, npl), jnp.float32)
        wu, bu = _padW(p["up"][i]["w"]), _padb(p["up"][i]["b"])
        if up == 0:
            h_u, h_fin = _gcn(Plvl, u, ones_l, ss[up], wu, bu,
                              n_valid=n, add=down[up], add2=org_h)
            hs_out.append(h_u)
            hs_out.append(h_fin)
        else:
            h_u = _gcn(Plvl, u, ones_l, ss[up], wu, bu,
                       n_valid=n, add=down[up])
            hs_out.append(h_u)
        cur = h_u

    hstack = jnp.stack([_pad_rows(x, _N) for x in hs_out], axis=1)
    ro = _readout(hstack)
    emb = jnp.concatenate([ro[:, s_, r, :_F] for r in (0, 1, 2)
                           for s_ in range(4)], axis=-1)
    b1 = jnp.broadcast_to(p["out1"]["b"][None, :], (8, _HIDDEN))
    b2 = jnp.broadcast_to(p["out2"]["b"][None, :], (8, _NCLS))
    return _classifier(emb, p["out1"]["w"], b1, p["out2"]["w"], b2)
